# baseline (speedup 1.0000x reference)
import functools

import jax
import jax.numpy as jnp
from jax import lax
from jax.experimental import pallas as pl
from jax.experimental.pallas import tpu as pltpu
from jax.experimental.pallas import tpu_sc as plsc

B = 16
S = 512
NOFF = 36
LN2 = 0.6931472
SQRT2 = 1.4142135


def _vlog(x):
    bits = plsc.bitcast(x, jnp.int32)
    e = (bits >> 23) - 127
    m = plsc.bitcast((bits & 0x007FFFFF) | 0x3F800000, jnp.float32)
    big = m > SQRT2
    m = jnp.where(big, m * 0.5, m)
    ef = (e + big.astype(jnp.int32)).astype(jnp.float32)
    t = (m - 1.0) / (m + 1.0)
    s = t * t
    p = (1.0 / 3.0) + s * ((1.0 / 5.0) + s * ((1.0 / 7.0) + s * (1.0 / 9.0)))
    lm = 2.0 * t + 2.0 * t * s * p
    return jnp.where(x == 0.0, -jnp.inf, ef * LN2 + lm)


def _sc_body(table_hbm, gt_hbm, out_hbm, gt_v, idx_v, vals_v,
             acc_t, acc_c, res_v, sem):
    is_lead = (lax.axis_index("c") == 0) & (lax.axis_index("s") == 0)

    @pl.when(is_lead)
    def _():
        pltpu.sync_copy(gt_hbm, gt_v)
        iota = lax.iota(jnp.int32, 16)
        zeros = jnp.zeros((16,), jnp.int32)
        g0 = gt_v[0, :]
        g1 = gt_v[1, :]

        def build(k, carry):
            oi = k // 6 - 3
            oj = k % 6 - 3
            pc0 = jnp.clip(g0 + oi, 0, S - 1)
            pc1 = jnp.clip(g1 + oj, 0, S - 1)
            idx_v[k // 8, pl.ds((k % 8) * 16, 16)] = iota * (S * S) + pc0 * S + pc1
            return carry

        lax.fori_loop(0, NOFF, build, 0)
        for t in range(4):
            idx_v[4, pl.ds(64 + 16 * t, 16)] = zeros

        copies = [
            pltpu.async_copy(table_hbm.at[idx_v.at[j]],
                             vals_v.at[pl.ds(j * 128, 128)], sem)
            for j in range(5)
        ]
        for c in copies:
            c.wait()

        g0f = g0.astype(jnp.float32)
        g1f = g1.astype(jnp.float32)
        s0 = jnp.sum(g0f)
        s1 = jnp.sum(g1f)
        q = jnp.sum(g0f * g0f + g1f * g1f)
        acc_t[...] = jnp.zeros((16,), jnp.float32)
        acc_c[...] = jnp.zeros((16,), jnp.float32)

        def combine(k, carry):
            oi = k // 6 - 3
            oj = k % 6 - 3
            p0 = g0 + oi
            p1 = g1 + oj
            valid = (p0 >= 0) & (p0 < S) & (p1 >= 0) & (p1 < S)
            yh = vals_v[pl.ds(16 * k, 16)]
            p0f = p0.astype(jnp.float32)
            p1f = p1.astype(jnp.float32)
            normsq = (16.0 * (p0f * p0f + p1f * p1f)
                      - 2.0 * (p0f * s0 + p1f * s1) + q)
            y = jnp.exp(-normsq / 5.0)
            d = yh - y
            d2 = d * d
            omy = 1.0 - y
            omy2 = omy * omy
            pos_t = -_vlog(yh) * d2
            neg_t = -_vlog(1.0 - yh) * (omy2 * omy2) * d2
            term = jnp.where(y == 1.0, pos_t, neg_t)
            vf = valid.astype(jnp.float32)
            acc_t[...] = acc_t[...] + term * vf
            acc_c[...] = acc_c[...] + vf
            return carry

        lax.fori_loop(0, NOFF, combine, 0)

        zf = jnp.zeros((16,), jnp.float32)
        res_v[...] = (zf + jnp.sum(acc_t[...])) / (zf + jnp.sum(acc_c[...]))
        pltpu.sync_copy(res_v, out_hbm)


_sc_loss = functools.partial(
    pl.kernel,
    out_type=jax.ShapeDtypeStruct((16,), jnp.float32),
    mesh=plsc.VectorSubcoreMesh(core_axis_name="c", subcore_axis_name="s"),
    compiler_params=pltpu.CompilerParams(needs_layout_passes=False),
    scratch_types=[
        pltpu.VMEM((2, 16), jnp.int32),
        pltpu.VMEM((5, 128), jnp.int32),
        pltpu.VMEM((640,), jnp.float32),
        pltpu.VMEM((16,), jnp.float32),
        pltpu.VMEM((16,), jnp.float32),
        pltpu.VMEM((16,), jnp.float32),
        pltpu.SemaphoreType.DMA,
    ],
)(_sc_body)


@jax.jit
def kernel(y_predict, gt_pos):
    table = y_predict.reshape(B * S * S)
    gt_cm = gt_pos.astype(jnp.int32).T.reshape(2, 16)
    return _sc_loss(table, gt_cm)[0]


# speedup vs baseline: 1.5656x; 1.5656x over previous
import functools

import jax
import jax.numpy as jnp
from jax import lax
from jax.experimental import pallas as pl
from jax.experimental.pallas import tpu as pltpu
from jax.experimental.pallas import tpu_sc as plsc

B = 16
S = 512
NOFF = 36
LN2 = 0.6931472
SQRT2 = 1.4142135


def _vlog(x):
    bits = plsc.bitcast(x, jnp.int32)
    e = (bits >> 23) - 127
    m = plsc.bitcast((bits & 0x007FFFFF) | 0x3F800000, jnp.float32)
    big = m > SQRT2
    m = jnp.where(big, m * 0.5, m)
    ef = (e + big.astype(jnp.int32)).astype(jnp.float32)
    t = (m - 1.0) / (m + 1.0)
    s = t * t
    p = (1.0 / 3.0) + s * ((1.0 / 5.0) + s * ((1.0 / 7.0) + s * (1.0 / 9.0)))
    lm = 2.0 * t + 2.0 * t * s * p
    return jnp.where(x == 0.0, -jnp.inf, ef * LN2 + lm)


def _sc_body(table_hbm, gt_hbm, out_hbm, gt_v, idx_v, rows_v,
             acc_t, acc_c, res_v, sem):
    is_lead = (lax.axis_index("c") == 0) & (lax.axis_index("s") == 0)

    @pl.when(is_lead)
    def _():
        pltpu.sync_copy(gt_hbm, gt_v)
        iota = lax.iota(jnp.int32, 16)
        zeros = jnp.zeros((16,), jnp.int32)
        g0 = gt_v[0, :]
        g1 = gt_v[1, :]

        def build(d, carry):
            pc0 = jnp.clip(g0 + (d - 3), 0, S - 1)
            idx_v[pl.ds(16 * d, 16)] = iota * S + pc0
            return carry

        lax.fori_loop(0, 6, build, 0)
        idx_v[pl.ds(96, 16)] = zeros
        idx_v[pl.ds(112, 16)] = zeros

        pltpu.async_copy(table_hbm.at[idx_v], rows_v, sem).wait()

        g0f = g0.astype(jnp.float32)
        g1f = g1.astype(jnp.float32)
        s0 = jnp.sum(g0f)
        s1 = jnp.sum(g1f)
        q = jnp.sum(g0f * g0f + g1f * g1f)
        acc_t[...] = jnp.zeros((16,), jnp.float32)
        acc_c[...] = jnp.zeros((16,), jnp.float32)

        def combine(k, carry):
            oi = k // 6 - 3
            oj = k % 6 - 3
            p0 = g0 + oi
            p1 = g1 + oj
            valid = (p0 >= 0) & (p0 < S) & (p1 >= 0) & (p1 < S)
            pc1 = jnp.clip(p1, 0, S - 1)
            yh = plsc.load_gather(rows_v, [16 * (k // 6) + iota, pc1])
            p0f = p0.astype(jnp.float32)
            p1f = p1.astype(jnp.float32)
            normsq = (16.0 * (p0f * p0f + p1f * p1f)
                      - 2.0 * (p0f * s0 + p1f * s1) + q)
            y = jnp.exp(-normsq / 5.0)
            d = yh - y
            d2 = d * d
            omy = 1.0 - y
            omy2 = omy * omy
            pos_t = -_vlog(yh) * d2
            neg_t = -_vlog(1.0 - yh) * (omy2 * omy2) * d2
            term = jnp.where(y == 1.0, pos_t, neg_t)
            vf = valid.astype(jnp.float32)
            acc_t[...] = acc_t[...] + term * vf
            acc_c[...] = acc_c[...] + vf
            return carry

        lax.fori_loop(0, NOFF, combine, 0)

        zf = jnp.zeros((16,), jnp.float32)
        res_v[...] = (zf + jnp.sum(acc_t[...])) / (zf + jnp.sum(acc_c[...]))
        pltpu.sync_copy(res_v, out_hbm)


_sc_loss = functools.partial(
    pl.kernel,
    out_type=jax.ShapeDtypeStruct((16,), jnp.float32),
    mesh=plsc.VectorSubcoreMesh(core_axis_name="c", subcore_axis_name="s"),
    compiler_params=pltpu.CompilerParams(needs_layout_passes=False),
    scratch_types=[
        pltpu.VMEM((2, 16), jnp.int32),
        pltpu.VMEM((128,), jnp.int32),
        pltpu.VMEM((128, S), jnp.float32),
        pltpu.VMEM((16,), jnp.float32),
        pltpu.VMEM((16,), jnp.float32),
        pltpu.VMEM((16,), jnp.float32),
        pltpu.SemaphoreType.DMA,
    ],
)(_sc_body)


@jax.jit
def kernel(y_predict, gt_pos):
    table = y_predict.reshape(B * S, S)
    gt_cm = gt_pos.astype(jnp.int32).T.reshape(2, 16)
    return _sc_loss(table, gt_cm)[0]


# speedup vs baseline: 1.8030x; 1.1516x over previous
import functools

import jax
import jax.numpy as jnp
from jax import lax
from jax.experimental import pallas as pl
from jax.experimental.pallas import tpu as pltpu
from jax.experimental.pallas import tpu_sc as plsc

B = 16
S = 512
NT = 6
LN2 = 0.6931472
SQRT2 = 1.4142135


def _vlog(x):
    bits = plsc.bitcast(x, jnp.int32)
    e = (bits >> 23) - 127
    m = plsc.bitcast((bits & 0x007FFFFF) | 0x3F800000, jnp.float32)
    big = m > SQRT2
    m = jnp.where(big, m * 0.5, m)
    ef = (e + big.astype(jnp.int32)).astype(jnp.float32)
    t = (m - 1.0) / (m + 1.0)
    s = t * t
    p = (1.0 / 3.0) + s * ((1.0 / 5.0) + s * ((1.0 / 7.0) + s * (1.0 / 9.0)))
    lm = 2.0 * t + 2.0 * t * s * p
    return jnp.where(x == 0.0, -jnp.inf, ef * LN2 + lm)


def _sc_body(table_hbm, gt_hbm, out_hbm, gt_v, idx_v, rows_v, acc_v, all_v,
             res_v, shared, sem):
    cid = lax.axis_index("c")
    sid = lax.axis_index("s")
    on_core0 = cid == 0

    @pl.when(on_core0 & (sid < NT))
    def _():
        pltpu.sync_copy(gt_hbm, gt_v)
        iota = lax.iota(jnp.int32, 16)
        g0 = plsc.load_gather(gt_v, [iota, jnp.zeros((16,), jnp.int32)])
        g1 = plsc.load_gather(gt_v, [iota, jnp.ones((16,), jnp.int32)])

        oi = sid - 3
        p0 = g0 + oi
        pc0 = jnp.clip(p0, 0, S - 1)
        idx_v[...] = iota * S + pc0
        pltpu.async_copy(table_hbm.at[idx_v], rows_v, sem).wait()

        g0f = g0.astype(jnp.float32)
        g1f = g1.astype(jnp.float32)
        s0 = jnp.sum(g0f)
        s1 = jnp.sum(g1f)
        q = jnp.sum(g0f * g0f + g1f * g1f)
        p0f = p0.astype(jnp.float32)
        row_q = 16.0 * (p0f * p0f) - 2.0 * (p0f * s0) + q
        v0 = (p0 >= 0) & (p0 < S)

        acc_t = jnp.zeros((16,), jnp.float32)
        acc_c = jnp.zeros((16,), jnp.float32)
        for j in range(6):
            p1 = g1 + (j - 3)
            valid = v0 & (p1 >= 0) & (p1 < S)
            pc1 = jnp.clip(p1, 0, S - 1)
            yh = plsc.load_gather(rows_v, [iota, pc1])
            p1f = p1.astype(jnp.float32)
            normsq = row_q + 16.0 * (p1f * p1f) - 2.0 * (p1f * s1)
            y = jnp.exp(-normsq / 5.0)
            d = yh - y
            d2 = d * d
            omy = 1.0 - y
            omy2 = omy * omy
            pos_t = -_vlog(yh) * d2
            neg_t = -_vlog(1.0 - yh) * (omy2 * omy2) * d2
            term = jnp.where(y == 1.0, pos_t, neg_t)
            vf = valid.astype(jnp.float32)
            acc_t = acc_t + term * vf
            acc_c = acc_c + vf

        acc_v[pl.ds(0, 16)] = acc_t
        acc_v[pl.ds(16, 16)] = acc_c
        pltpu.sync_copy(acc_v, shared.at[pl.ds(32 * sid, 32)])

    plsc.subcore_barrier()

    @pl.when(on_core0 & (sid == 0))
    def _():
        pltpu.sync_copy(shared, all_v)
        tv = all_v[pl.ds(0, 16)]
        cv = all_v[pl.ds(16, 16)]
        for i in range(1, NT):
            tv = tv + all_v[pl.ds(32 * i, 16)]
            cv = cv + all_v[pl.ds(32 * i + 16, 16)]
        zf = jnp.zeros((16,), jnp.float32)
        res_v[...] = (zf + jnp.sum(tv)) / (zf + jnp.sum(cv))
        pltpu.sync_copy(res_v, out_hbm)


_sc_loss = functools.partial(
    pl.kernel,
    out_type=jax.ShapeDtypeStruct((16,), jnp.float32),
    mesh=plsc.VectorSubcoreMesh(core_axis_name="c", subcore_axis_name="s"),
    compiler_params=pltpu.CompilerParams(needs_layout_passes=False),
    scratch_types=[
        pltpu.VMEM((16, 2), jnp.int32),
        pltpu.VMEM((16,), jnp.int32),
        pltpu.VMEM((16, S), jnp.float32),
        pltpu.VMEM((32,), jnp.float32),
        pltpu.VMEM((32 * NT,), jnp.float32),
        pltpu.VMEM((16,), jnp.float32),
        pltpu.VMEM_SHARED((32 * NT,), jnp.float32),
        pltpu.SemaphoreType.DMA,
    ],
)(_sc_body)


@jax.jit
def kernel(y_predict, gt_pos):
    table = y_predict.reshape(B * S, S)
    return _sc_loss(table, gt_pos.astype(jnp.int32))[0]


# speedup vs baseline: 1.9216x; 1.0658x over previous
import functools

import jax
import jax.numpy as jnp
from jax import lax
from jax.experimental import pallas as pl
from jax.experimental.pallas import tpu as pltpu
from jax.experimental.pallas import tpu_sc as plsc

B = 16
S = 512
NT = 6
LN2 = 0.6931472
SQRT2 = 1.4142135


def _vlog(x):
    bits = plsc.bitcast(x, jnp.int32)
    e = (bits >> 23) - 127
    m = plsc.bitcast((bits & 0x007FFFFF) | 0x3F800000, jnp.float32)
    big = m > SQRT2
    m = jnp.where(big, m * 0.5, m)
    ef = (e + big.astype(jnp.int32)).astype(jnp.float32)
    t = (m - 1.0) / (m + 1.0)
    s = t * t
    p = (1.0 / 3.0) + s * ((1.0 / 5.0) + s * ((1.0 / 7.0) + s * (1.0 / 9.0)))
    lm = 2.0 * t + 2.0 * t * s * p
    return jnp.where(x == 0.0, -jnp.inf, ef * LN2 + lm)


def _sc_body(table_hbm, gt_hbm, out_hbm, gt_v, idx_v, rows_v, acc_v, all_v,
             res_v, shared, sem):
    cid = lax.axis_index("c")
    sid = lax.axis_index("s")
    on_core0 = cid == 0

    @pl.when(on_core0 & (sid < NT))
    def _():
        pltpu.sync_copy(gt_hbm, gt_v)
        iota = lax.iota(jnp.int32, 16)
        g0 = plsc.load_gather(gt_v, [iota, jnp.zeros((16,), jnp.int32)])
        g1 = plsc.load_gather(gt_v, [iota, jnp.ones((16,), jnp.int32)])

        oi = sid - 3
        p0 = g0 + oi
        pc0 = jnp.clip(p0, 0, S - 1)
        idx_v[...] = iota * S + pc0
        pltpu.async_copy(table_hbm.at[idx_v], rows_v, sem).wait()

        g0f = g0.astype(jnp.float32)
        g1f = g1.astype(jnp.float32)
        s0 = jnp.sum(g0f)
        s1 = jnp.sum(g1f)
        q = jnp.sum(g0f * g0f + g1f * g1f)
        p0f = p0.astype(jnp.float32)
        row_q = 16.0 * (p0f * p0f) - 2.0 * (p0f * s0) + q
        v0 = (p0 >= 0) & (p0 < S)

        acc_t = jnp.zeros((16,), jnp.float32)
        acc_c = jnp.zeros((16,), jnp.float32)
        for j in range(6):
            p1 = g1 + (j - 3)
            valid = v0 & (p1 >= 0) & (p1 < S)
            pc1 = jnp.clip(p1, 0, S - 1)
            yh = plsc.load_gather(rows_v, [iota, pc1])
            p1f = p1.astype(jnp.float32)
            normsq = row_q + 16.0 * (p1f * p1f) - 2.0 * (p1f * s1)
            y = jnp.exp(-normsq / 5.0)
            d = yh - y
            d2 = d * d
            omy = 1.0 - y
            omy2 = omy * omy
            pos_t = -_vlog(yh) * d2
            neg_t = -_vlog(1.0 - yh) * (omy2 * omy2) * d2
            term = jnp.where(y == 1.0, pos_t, neg_t)
            vf = valid.astype(jnp.float32)
            acc_t = acc_t + term * vf
            acc_c = acc_c + vf

        acc_v[pl.ds(0, 16)] = acc_t
        acc_v[pl.ds(16, 16)] = acc_c
        pltpu.sync_copy(acc_v, shared.at[pl.ds(32 * sid, 32)])

    plsc.subcore_barrier()

    @pl.when(on_core0 & (sid == 0))
    def _():
        pltpu.sync_copy(shared, all_v)
        tv = all_v[pl.ds(0, 16)]
        cv = all_v[pl.ds(16, 16)]
        for i in range(1, NT):
            tv = tv + all_v[pl.ds(32 * i, 16)]
            cv = cv + all_v[pl.ds(32 * i + 16, 16)]
        zf = jnp.zeros((16,), jnp.float32)
        res_v[...] = (zf + jnp.sum(tv)) / (zf + jnp.sum(cv))
        pltpu.sync_copy(res_v.at[pl.ds(0, 1)], out_hbm)


_sc_loss = functools.partial(
    pl.kernel,
    out_type=jax.ShapeDtypeStruct((1,), jnp.float32),
    mesh=plsc.VectorSubcoreMesh(core_axis_name="c", subcore_axis_name="s",
                                num_cores=1),
    compiler_params=pltpu.CompilerParams(needs_layout_passes=False),
    scratch_types=[
        pltpu.VMEM((16, 2), jnp.int32),
        pltpu.VMEM((16,), jnp.int32),
        pltpu.VMEM((16, S), jnp.float32),
        pltpu.VMEM((32,), jnp.float32),
        pltpu.VMEM((32 * NT,), jnp.float32),
        pltpu.VMEM((16,), jnp.float32),
        pltpu.VMEM_SHARED((32 * NT,), jnp.float32),
        pltpu.SemaphoreType.DMA,
    ],
)(_sc_body)


@jax.jit
def kernel(y_predict, gt_pos):
    table = y_predict.reshape(B * S, S)
    return _sc_loss(table, gt_pos.astype(jnp.int32)).reshape(())
